# baseline (speedup 1.0000x reference)
import jax
import jax.numpy as jnp
from jax.experimental import pallas as pl
import jax.experimental.pallas.tpu as pltpu

_B, _C, _H, _W = 16, 256, 32, 32
_HW = _H * _W
_N = _B * _HW
_COMMIT = 0.25
_SCALE = (1.0 + _COMMIT) / (_N * _C)
_BB = 4
_STEPS = _B // _BB


def _vq_body(x_ref, w_ref, out_ref, sidx_ref, cidx_ref, loss_ref,
             tpl_v, zeros_v, sems):
    i = pl.program_id(0)

    def _out_copies():
        cps = [pltpu.make_async_copy(tpl_v, out_ref.at[b], sems.at[b % 4])
               for b in range(_B)]
        cps.append(pltpu.make_async_copy(zeros_v, sidx_ref, sems.at[0]))
        cps.append(pltpu.make_async_copy(zeros_v, cidx_ref, sems.at[1]))
        return cps

    @pl.when(i == 0)
    def _prep():
        tpl_v[...] = jnp.broadcast_to(w_ref[...].reshape(_C, 1), (_C, _HW))
        zeros_v[...] = jnp.zeros((_NS_IDX, _HW), jnp.int32)
        loss_ref[...] = jnp.zeros((1, 1), jnp.float32)

    d = x_ref[...] - w_ref[...]
    part = jnp.sum(d * d) * _SCALE
    loss_ref[...] += part.reshape(1, 1)

    @pl.when(i == _STEPS - 1)
    def _write_out():
        for cp in _out_copies():
            cp.start()
        for cp in _out_copies():
            cp.wait()


_NS_IDX = 16


def kernel(inputs, W_shape, W_color):
    x = inputs.reshape(_B, _C, _HW)
    w_cat = jnp.concatenate([W_shape[0], W_color[0]]).reshape(1, _C, 1)

    out, sidx, cidx, loss = pl.pallas_call(
        _vq_body,
        grid=(_STEPS,),
        in_specs=[
            pl.BlockSpec((_BB, _C, _HW), lambda i: (i, 0, 0)),
            pl.BlockSpec((1, _C, 1), lambda i: (0, 0, 0)),
        ],
        out_specs=[
            pl.BlockSpec(memory_space=pltpu.MemorySpace.HBM),
            pl.BlockSpec(memory_space=pltpu.MemorySpace.HBM),
            pl.BlockSpec(memory_space=pltpu.MemorySpace.HBM),
            pl.BlockSpec((1, 1), lambda i: (0, 0)),
        ],
        out_shape=[
            jax.ShapeDtypeStruct((_B, _C, _HW), jnp.float32),
            jax.ShapeDtypeStruct((_NS_IDX, _HW), jnp.int32),
            jax.ShapeDtypeStruct((_NS_IDX, _HW), jnp.int32),
            jax.ShapeDtypeStruct((1, 1), jnp.float32),
        ],
        scratch_shapes=[
            pltpu.VMEM((_C, _HW), jnp.float32),
            pltpu.VMEM((_NS_IDX, _HW), jnp.int32),
            pltpu.SemaphoreType.DMA((4,)),
        ],
    )(x, w_cat)

    quantized = out.reshape(_B, _C, _H, _W)
    vq_loss = loss[0, 0]
    rep_loss = jnp.float32(0.0)
    return quantized, vq_loss, rep_loss, sidx.reshape(_N), cidx.reshape(_N)
